# Initial kernel scaffold; baseline (speedup 1.0000x reference)
import jax
import jax.numpy as jnp
from jax.experimental import pallas as pl


def kernel(pharm_h, prot_h, pharm_x, prot_x, t, batch_pharm, batch_prot, edge_ff, edge_pf, edge_fp, edge_pp, params):
    raise NotImplementedError("write your pallas kernel here")


# speedup vs baseline: 9.6310x; 9.6310x over previous
import functools

import jax
import jax.numpy as jnp
import numpy as np
from jax.experimental import pallas as pl
from jax.experimental.pallas import tpu as pltpu

N_PHARM = 10000
N_PROT = 40000
E = 200000
B = 16
S = 64
V = 16
RBF = 32
RBF_DMAX = 20.0
N_CONVS = 3
SIG = RBF_DMAX / RBF

_EDGE_BLK = 2000


def _ln(x, g, b):
    m = jnp.mean(x, -1, keepdims=True)
    v = jnp.var(x, -1, keepdims=True)
    return (x - m) / jnp.sqrt(v + 1e-5) * g + b


def _edge_body(ssrc_ref, sdst_ref, xd_ref, vec_ref,
               w1_ref, w2_ref, w3_ref, b_ref, r_ref, q_ref, mu_ref,
               ms_ref, mv_ref):
    ssrc = ssrc_ref[...]
    sdst = sdst_ref[...]
    xd = xd_ref[...]
    d2 = jnp.sum(xd * xd, axis=1, keepdims=True) + 1e-8
    d = jnp.sqrt(d2)
    xdn = xd / d
    mu = mu_ref[...]
    rbf = jnp.exp(-jnp.square((d - mu) / SIG))

    w1 = w1_ref[...]
    w2 = w2_ref[...]
    w3 = w3_ref[...]
    bias = b_ref[...]
    pre = (jnp.dot(ssrc, w1, preferred_element_type=jnp.float32)
           + jnp.dot(sdst, w2, preferred_element_type=jnp.float32)
           + jnp.dot(rbf, w3, preferred_element_type=jnp.float32)
           + bias)
    ms = pre[:, :S] * jax.nn.sigmoid(pre[:, :S])
    gv = jax.nn.sigmoid(pre[:, S:S + V])
    gx = jax.nn.sigmoid(pre[:, S + V:S + 2 * V])
    gv48 = jnp.dot(gv, r_ref[...], preferred_element_type=jnp.float32)
    gx48 = jnp.dot(gx, r_ref[...], preferred_element_type=jnp.float32)
    xdn48 = jnp.dot(xdn, q_ref[...], preferred_element_type=jnp.float32)
    ms_ref[...] = ms
    mv_ref[...] = vec_ref[...] * gv48 + xdn48 * gx48


@functools.partial(jax.jit, static_argnames=())
def _edge_messages(ssrc_g, sdst_g, xd, vec_g, w1, w2, w3, bias):
    ne = ssrc_g.shape[0]
    grid = ne // _EDGE_BLK
    r = np.zeros((V, 3 * V), np.float32)
    for v in range(V):
        r[v, 3 * v:3 * v + 3] = 1.0
    q = np.zeros((4, 3 * V), np.float32)
    for v in range(V):
        for i in range(3):
            q[i, 3 * v + i] = 1.0
    blk = lambda j: pl.BlockSpec((_EDGE_BLK, None), lambda i: (i, 0))
    full = lambda shape: pl.BlockSpec(shape, lambda i: tuple(0 for _ in shape))
    ms, mv = pl.pallas_call(
        _edge_body,
        grid=(grid,),
        in_specs=[
            pl.BlockSpec((_EDGE_BLK, S), lambda i: (i, 0)),
            pl.BlockSpec((_EDGE_BLK, S), lambda i: (i, 0)),
            pl.BlockSpec((_EDGE_BLK, 4), lambda i: (i, 0)),
            pl.BlockSpec((_EDGE_BLK, 3 * V), lambda i: (i, 0)),
            full((S, S + 2 * V)),
            full((S, S + 2 * V)),
            full((RBF, S + 2 * V)),
            full((1, S + 2 * V)),
            full((V, 3 * V)),
            full((4, 3 * V)),
            full((1, RBF)),
        ],
        out_specs=[
            pl.BlockSpec((_EDGE_BLK, S), lambda i: (i, 0)),
            pl.BlockSpec((_EDGE_BLK, 3 * V), lambda i: (i, 0)),
        ],
        out_shape=[
            jax.ShapeDtypeStruct((ne, S), jnp.float32),
            jax.ShapeDtypeStruct((ne, 3 * V), jnp.float32),
        ],
    )(ssrc_g, sdst_g, xd, vec_g, w1, w2, w3, bias,
      jnp.asarray(r), jnp.asarray(q),
      jnp.asarray(np.linspace(0.0, RBF_DMAX, RBF, dtype=np.float32))[None])
    return ms, mv


def _conv_et(cp, s_src, s_dst, vec_src, src, dst, pos_src, pos_dst, ndst):
    w1 = jnp.concatenate([cp['msg']['w'][:S], cp['vgate']['w'][:S],
                          cp['xgate']['w'][:S]], axis=1)
    w2 = jnp.concatenate([cp['msg']['w'][S:2 * S], cp['vgate']['w'][S:2 * S],
                          cp['xgate']['w'][S:2 * S]], axis=1)
    w3 = jnp.concatenate([cp['msg']['w'][2 * S:], cp['vgate']['w'][2 * S:],
                          cp['xgate']['w'][2 * S:]], axis=1)
    bias = jnp.concatenate([cp['msg']['b'], cp['vgate']['b'],
                            cp['xgate']['b']]).reshape(1, -1)
    xd = pos_src[src] - pos_dst[dst]
    xd4 = jnp.pad(xd, ((0, 0), (0, 1)))
    ssrc_g = s_src[src]
    sdst_g = s_dst[dst]
    vec_g = vec_src.reshape(-1, 3 * V)[src]
    ms, mv = _edge_messages(ssrc_g, sdst_g, xd4, vec_g, w1, w2, w3, bias)
    agg_s = jax.ops.segment_sum(ms, dst, ndst)
    agg_v = jax.ops.segment_sum(mv, dst, ndst)
    cnt = jax.ops.segment_sum(jnp.ones((src.shape[0],), jnp.float32), dst, ndst)
    den = jnp.maximum(cnt, 1.0)
    return agg_s / den[:, None], (agg_v / den[:, None]).reshape(ndst, V, 3)


def _gvp(p, s, Vv, last):
    Vh = jnp.einsum('nvi,vh->nhi', Vv, p['wh'])
    norms = jnp.sqrt(jnp.sum(Vh * Vh, -1) + 1e-8)
    s_pre = jnp.concatenate([s, norms], -1) @ p['ws']['w'] + p['ws']['b']
    s_out = jax.nn.silu(s_pre)
    Vout = jnp.einsum('nhi,hv->nvi', Vh, p['wv'])
    gate = s_out @ p['wg']['w'] + p['wg']['b']
    if not last:
        gate = jax.nn.sigmoid(gate)
    return s_out, Vout * gate[..., None]


def kernel(pharm_h, prot_h, pharm_x, prot_x, t, batch_pharm, batch_prot,
           edge_ff, edge_pf, edge_fp, edge_pp, params):
    s = {}
    s['pharm'] = params['emb_pharm'][pharm_h]
    s['prot'] = params['emb_prot'][prot_h]
    batch = {'pharm': batch_pharm, 'prot': batch_prot}
    for nt in ['pharm', 'prot']:
        temb = t[batch[nt]][:, None]
        h = jnp.concatenate([s[nt], temb], -1)
        ep = params['embed_' + nt]
        h = jax.nn.silu(h @ ep['l1']['w'] + ep['l1']['b'])
        h = jax.nn.silu(h @ ep['l2']['w'] + ep['l2']['b'])
        s[nt] = _ln(h, ep['ln_g'], ep['ln_b'])
    pos = {'pharm': pharm_x, 'prot': prot_x}
    vec = {'pharm': jnp.zeros((pharm_x.shape[0], V, 3)),
           'prot': jnp.zeros((prot_x.shape[0], V, 3))}
    etypes = {'ff': ('pharm', 'pharm', edge_ff), 'pf': ('prot', 'pharm', edge_pf),
              'fp': ('pharm', 'prot', edge_fp), 'pp': ('prot', 'prot', edge_pp)}
    for ci, cp in enumerate(params['convs']):
        agg_s = {nt: jnp.zeros_like(s[nt]) for nt in s}
        agg_v = {nt: jnp.zeros_like(vec[nt]) for nt in vec}
        for et in ['ff', 'pf', 'fp', 'pp']:
            snt, dnt, eidx = etypes[et]
            a_s, a_v = _conv_et(cp[et], s[snt], s[dnt], vec[snt], eidx[0],
                                eidx[1], pos[snt], pos[dnt], s[dnt].shape[0])
            agg_s[dnt] = agg_s[dnt] + a_s
            agg_v[dnt] = agg_v[dnt] + a_v
        for nt in ['pharm', 'prot']:
            up = cp['upd_' + nt]
            s[nt] = _ln(s[nt] + agg_s[nt] @ up['l']['w'] + up['l']['b'],
                        up['ln_g'], up['ln_b'])
            vec[nt] = vec[nt] + agg_v[nt]
        gvps = params['updaters'][ci]
        hs, hv = s['pharm'], vec['pharm']
        for gi, gp in enumerate(gvps):
            hs, hv = _gvp(gp, hs, hv, gi == 2)
        pos['pharm'] = pos['pharm'] + hv[:, 0, :]
    r = params['readout']
    logits = (jax.nn.silu(s['pharm'] @ r['l1']['w'] + r['l1']['b'])
              @ r['l2']['w'] + r['l2']['b'])
    return logits, pos['pharm']


# speedup vs baseline: 11.6210x; 1.2066x over previous
import functools

import jax
import jax.numpy as jnp
import numpy as np
from jax import lax
from jax.experimental import pallas as pl
from jax.experimental.pallas import tpu as pltpu
from jax.experimental.pallas import tpu_sc as plsc

N_PHARM = 10000
N_PROT = 40000
E = 200000
B = 16
S = 64
V = 16
RBF = 32
RBF_DMAX = 20.0
N_CONVS = 3
SIG = RBF_DMAX / RBF

_EDGE_BLK = 2000

_NC = 2
_NS = 16
_NW = _NC * _NS
_GAT_K = 400
_SEG_K = 80


def _sc_count(dst, ndst):
    ones = jnp.ones((E, S), jnp.float32)
    zeros = jnp.zeros((E, S), jnp.float32)
    agg_s, _ = _sc_segsum_pair(ones, zeros, zeros, zeros, dst, dst, ndst)
    return agg_s[:, 0]


def _sc_gather_et(t_a, t_v, t_b, src, dst):
    ne = src.shape[0]
    nb = ne // _GAT_K
    mesh = plsc.VectorSubcoreMesh(core_axis_name="c", subcore_axis_name="s",
                                  num_cores=_NC, num_subcores=_NS)

    def body(ta_ref, tv_ref, tb_ref, src_ref, dst_ref,
             oa_ref, ov_ref, ob_ref, sidx, didx, arows, vrows, brows, sem):
        cid = lax.axis_index("c")
        sid = lax.axis_index("s")
        wid = sid * _NC + cid

        def batch(bi, carry):
            b = wid + bi * _NW

            @pl.when(b < nb)
            def _():
                e0 = b * _GAT_K
                pltpu.sync_copy(src_ref.at[pl.ds(e0, _GAT_K)], sidx)
                pltpu.sync_copy(dst_ref.at[pl.ds(e0, _GAT_K)], didx)
                ca = pltpu.async_copy(ta_ref.at[sidx], arows, sem)
                ca.wait()
                cv = pltpu.async_copy(tv_ref.at[sidx], vrows, sem)
                cv.wait()
                cb = pltpu.async_copy(tb_ref.at[didx], brows, sem)
                cb.wait()
                pltpu.sync_copy(arows, oa_ref.at[pl.ds(e0, _GAT_K)])
                pltpu.sync_copy(vrows, ov_ref.at[pl.ds(e0, _GAT_K)])
                pltpu.sync_copy(brows, ob_ref.at[pl.ds(e0, _GAT_K)])
            return carry

        lax.fori_loop(0, (nb + _NW - 1) // _NW, batch, 0)

    wa = t_a.shape[1]
    wv = t_v.shape[1]
    wb = t_b.shape[1]
    return pl.kernel(
        body,
        out_type=[jax.ShapeDtypeStruct((ne, wa), jnp.float32),
                  jax.ShapeDtypeStruct((ne, wv), jnp.float32),
                  jax.ShapeDtypeStruct((ne, wb), jnp.float32)],
        mesh=mesh,
        scratch_types=[pltpu.VMEM((_GAT_K,), jnp.int32),
                       pltpu.VMEM((_GAT_K,), jnp.int32),
                       pltpu.VMEM((_GAT_K, wa), jnp.float32),
                       pltpu.VMEM((_GAT_K, wv), jnp.float32),
                       pltpu.VMEM((_GAT_K, wb), jnp.float32),
                       pltpu.SemaphoreType.DMA],
        compiler_params=pltpu.CompilerParams(use_tc_tiling_on_sc=False),
    )(t_a, t_v, t_b, src, dst)


def _sc_segsum_pair(ms_a, mv_a, ms_b, mv_b, dst_a, dst_b, ndst):
    agg_s = (jax.ops.segment_sum(ms_a, dst_a, ndst)
             + jax.ops.segment_sum(ms_b, dst_b, ndst))
    agg_v = (jax.ops.segment_sum(mv_a, dst_a, ndst)
             + jax.ops.segment_sum(mv_b, dst_b, ndst))
    return agg_s, agg_v[:, :3 * V]


def _ln(x, g, b):
    m = jnp.mean(x, -1, keepdims=True)
    v = jnp.var(x, -1, keepdims=True)
    return (x - m) / jnp.sqrt(v + 1e-5) * g + b


def _edge_body(oa_ref, ob_ref, ov_ref, w3_ref, r_ref, q_ref, mu_ref,
               ms_ref, mv_ref):
    comb = oa_ref[...] + ob_ref[...]
    xd = comb[:, 96:100]
    d2 = jnp.sum(xd * xd, axis=1, keepdims=True) + 1e-8
    d = jnp.sqrt(d2)
    xdn = xd / d
    mu = mu_ref[...]
    rbf = jnp.exp(-jnp.square((d - mu) / SIG))
    pre = (comb[:, :S + 2 * V]
           + jnp.dot(rbf, w3_ref[...], preferred_element_type=jnp.float32))
    ms = pre[:, :S] * jax.nn.sigmoid(pre[:, :S])
    gv = jax.nn.sigmoid(pre[:, S:S + V])
    gx = jax.nn.sigmoid(pre[:, S + V:S + 2 * V])
    gv48 = jnp.dot(gv, r_ref[...], preferred_element_type=jnp.float32)
    gx48 = jnp.dot(gx, r_ref[...], preferred_element_type=jnp.float32)
    xdn48 = jnp.dot(xdn, q_ref[...], preferred_element_type=jnp.float32)
    invd = comb[:, 100:101]
    ms_ref[...] = ms * invd
    mv = (ov_ref[...] * gv48 + xdn48 * gx48) * invd
    mv_ref[...] = jnp.concatenate(
        [mv, jnp.zeros((mv.shape[0], S - 3 * V), jnp.float32)], axis=1)


def _edge_messages(o_a, o_b, o_v, w3):
    ne = o_a.shape[0]
    grid = ne // _EDGE_BLK
    r = np.zeros((V, 3 * V), np.float32)
    for v in range(V):
        r[v, 3 * v:3 * v + 3] = 1.0
    q = np.zeros((4, 3 * V), np.float32)
    for v in range(V):
        for i in range(3):
            q[i, 3 * v + i] = 1.0
    full = lambda shape: pl.BlockSpec(shape, lambda i: tuple(0 for _ in shape))
    ms, mv = pl.pallas_call(
        _edge_body,
        grid=(grid,),
        in_specs=[
            pl.BlockSpec((_EDGE_BLK, 112), lambda i: (i, 0)),
            pl.BlockSpec((_EDGE_BLK, 112), lambda i: (i, 0)),
            pl.BlockSpec((_EDGE_BLK, 3 * V), lambda i: (i, 0)),
            full((RBF, S + 2 * V)),
            full((V, 3 * V)),
            full((4, 3 * V)),
            full((1, RBF)),
        ],
        out_specs=[
            pl.BlockSpec((_EDGE_BLK, S), lambda i: (i, 0)),
            pl.BlockSpec((_EDGE_BLK, S), lambda i: (i, 0)),
        ],
        out_shape=[
            jax.ShapeDtypeStruct((ne, S), jnp.float32),
            jax.ShapeDtypeStruct((ne, S), jnp.float32),
        ],
    )(o_a, o_b, o_v, w3,
      jnp.asarray(r), jnp.asarray(q),
      jnp.asarray(np.linspace(0.0, RBF_DMAX, RBF, dtype=np.float32))[None])
    return ms, mv


def _conv_et_messages(cp, s_src, s_dst, vec_src, src, dst, pos_src, pos_dst,
                      invd_dst):
    w1 = jnp.concatenate([cp['msg']['w'][:S], cp['vgate']['w'][:S],
                          cp['xgate']['w'][:S]], axis=1)
    w2 = jnp.concatenate([cp['msg']['w'][S:2 * S], cp['vgate']['w'][S:2 * S],
                          cp['xgate']['w'][S:2 * S]], axis=1)
    w3 = jnp.concatenate([cp['msg']['w'][2 * S:], cp['vgate']['w'][2 * S:],
                          cp['xgate']['w'][2 * S:]], axis=1)
    bias = jnp.concatenate([cp['msg']['b'], cp['vgate']['b'],
                            cp['xgate']['b']]).reshape(1, -1)
    ns = s_src.shape[0]
    nd = s_dst.shape[0]
    t_a = jnp.concatenate([s_src @ w1, pos_src, jnp.zeros((ns, 13))], axis=1)
    t_b = jnp.concatenate([
        s_dst @ w2 + bias, -pos_dst, jnp.zeros((nd, 1)), invd_dst[:, None],
        jnp.zeros((nd, 11))], axis=1)
    t_v = vec_src.reshape(-1, 3 * V)
    o_a, o_v, o_b = _sc_gather_et(t_a, t_v, t_b, src, dst)
    return _edge_messages(o_a, o_b, o_v, w3)


def _gvp(p, s, Vv, last):
    Vh = jnp.einsum('nvi,vh->nhi', Vv, p['wh'])
    norms = jnp.sqrt(jnp.sum(Vh * Vh, -1) + 1e-8)
    s_pre = jnp.concatenate([s, norms], -1) @ p['ws']['w'] + p['ws']['b']
    s_out = jax.nn.silu(s_pre)
    Vout = jnp.einsum('nhi,hv->nvi', Vh, p['wv'])
    gate = s_out @ p['wg']['w'] + p['wg']['b']
    if not last:
        gate = jax.nn.sigmoid(gate)
    return s_out, Vout * gate[..., None]


def kernel(pharm_h, prot_h, pharm_x, prot_x, t, batch_pharm, batch_prot,
           edge_ff, edge_pf, edge_fp, edge_pp, params):
    def _onehot_lookup(idx, table):
        oh = (idx[:, None] == jnp.arange(table.shape[0])[None, :])
        return oh.astype(jnp.float32) @ table

    s = {}
    s['pharm'] = _onehot_lookup(pharm_h, params['emb_pharm'])
    s['prot'] = _onehot_lookup(prot_h, params['emb_prot'])
    batch = {'pharm': batch_pharm, 'prot': batch_prot}
    for nt in ['pharm', 'prot']:
        temb = _onehot_lookup(batch[nt], t[:, None])
        h = jnp.concatenate([s[nt], temb], -1)
        ep = params['embed_' + nt]
        h = jax.nn.silu(h @ ep['l1']['w'] + ep['l1']['b'])
        h = jax.nn.silu(h @ ep['l2']['w'] + ep['l2']['b'])
        s[nt] = _ln(h, ep['ln_g'], ep['ln_b'])
    pos = {'pharm': pharm_x, 'prot': prot_x}
    vec = {'pharm': jnp.zeros((pharm_x.shape[0], V, 3)),
           'prot': jnp.zeros((prot_x.shape[0], V, 3))}
    etypes = {'ff': ('pharm', 'pharm', edge_ff), 'pf': ('prot', 'pharm', edge_pf),
              'fp': ('pharm', 'prot', edge_fp), 'pp': ('prot', 'prot', edge_pp)}
    invd_n = {}
    for et, (snt, dnt, eidx) in etypes.items():
        ndst = N_PHARM if dnt == 'pharm' else N_PROT
        cnt = _sc_count(eidx[1], ndst)
        invd_n[et] = 1.0 / jnp.maximum(cnt, 1.0)

    convs_x = jax.tree.map(lambda *xs: jnp.stack(xs), *params['convs'])
    ups_x = [jax.tree.map(lambda *xs: jnp.stack(xs),
                          *[params['updaters'][ci][gi]
                            for ci in range(N_CONVS)])
             for gi in range(3)]

    def conv_body(carry, xs):
        s_ph, s_pr, v_ph, v_pr, p_ph = carry
        cp, g0, g1, g2 = xs
        s_c = {'pharm': s_ph, 'prot': s_pr}
        vec_c = {'pharm': v_ph, 'prot': v_pr}
        pos_c = {'pharm': p_ph, 'prot': prot_x}
        ms = {}
        mv = {}
        for et in ['ff', 'pf', 'fp', 'pp']:
            snt, dnt, eidx = etypes[et]
            ms[et], mv[et] = _conv_et_messages(
                cp[et], s_c[snt], s_c[dnt], vec_c[snt], eidx[0], eidx[1],
                pos_c[snt], pos_c[dnt], invd_n[et])
        agg_s = {}
        agg_v = {}
        agg_s['pharm'], agg_v['pharm'] = _sc_segsum_pair(
            ms['ff'], mv['ff'], ms['pf'], mv['pf'],
            edge_ff[1], edge_pf[1], N_PHARM)
        agg_s['prot'], agg_v['prot'] = _sc_segsum_pair(
            ms['fp'], mv['fp'], ms['pp'], mv['pp'],
            edge_fp[1], edge_pp[1], N_PROT)
        for nt in ['pharm', 'prot']:
            up = cp['upd_' + nt]
            s_c[nt] = _ln(s_c[nt] + agg_s[nt] @ up['l']['w'] + up['l']['b'],
                          up['ln_g'], up['ln_b'])
            vec_c[nt] = vec_c[nt] + agg_v[nt].reshape(-1, V, 3)
        hs, hv = s_c['pharm'], vec_c['pharm']
        for gi, gp in enumerate((g0, g1, g2)):
            hs, hv = _gvp(gp, hs, hv, gi == 2)
        p_ph = pos_c['pharm'] + hv[:, 0, :]
        return (s_c['pharm'], s_c['prot'], vec_c['pharm'], vec_c['prot'],
                p_ph), None

    carry = (s['pharm'], s['prot'], vec['pharm'], vec['prot'], pos['pharm'])
    carry, _ = lax.scan(conv_body, carry, (convs_x, *ups_x))
    s['pharm'], s['prot'], vec['pharm'], vec['prot'], pos['pharm'] = carry
    r = params['readout']
    logits = (jax.nn.silu(s['pharm'] @ r['l1']['w'] + r['l1']['b'])
              @ r['l2']['w'] + r['l2']['b'])
    return logits, pos['pharm']
